# baseline (speedup 1.0000x reference)
import functools

import jax
import jax.numpy as jnp
from jax import lax
from jax.experimental import pallas as pl
from jax.experimental.pallas import tpu as pltpu
from jax.experimental.pallas import tpu_sc as plsc

_LANES = 16


def _sc_build_rows(inds_reac, coeffs_buf, R):
    NSEL = coeffs_buf.shape[0]
    mesh = plsc.VectorSubcoreMesh(core_axis_name="c", subcore_axis_name="s")

    @functools.partial(
        pl.kernel,
        mesh=mesh,
        compiler_params=pltpu.CompilerParams(needs_layout_passes=False),
        out_type=[
            jax.ShapeDtypeStruct((R,), jnp.float32),
            jax.ShapeDtypeStruct((R,), jnp.float32),
        ],
        scratch_types=[
            pltpu.VMEM((NSEL,), jnp.int32),
            pltpu.VMEM((NSEL,), jnp.float32),
            pltpu.VMEM((R,), jnp.float32),
            pltpu.VMEM((R,), jnp.float32),
        ],
    )
    def sc_kernel(inds_hbm, buf_hbm, vals_out, mask_out,
                  inds_v, buf_v, vals_v, mask_v):
        cid = lax.axis_index("c")
        sid = lax.axis_index("s")

        @pl.when(jnp.logical_and(cid == 0, sid == 0))
        def _():
            pltpu.sync_copy(inds_hbm, inds_v)
            pltpu.sync_copy(buf_hbm, buf_v)

            zeros = jnp.zeros((_LANES,), jnp.float32)

            def zero_body(i, carry):
                vals_v[pl.ds(i * _LANES, _LANES)] = zeros
                mask_v[pl.ds(i * _LANES, _LANES)] = zeros
                return carry

            lax.fori_loop(0, R // _LANES, zero_body, 0)

            ones = jnp.ones((_LANES,), jnp.float32)

            def scatter_body(j, carry):
                idx = inds_v[pl.ds(j * _LANES, _LANES)]
                val = buf_v[pl.ds(j * _LANES, _LANES)]
                plsc.store_scatter(vals_v, [idx], val)
                plsc.store_scatter(mask_v, [idx], ones)
                return carry

            lax.fori_loop(0, NSEL // _LANES, scatter_body, 0)

            pltpu.sync_copy(vals_v, vals_out)
            pltpu.sync_copy(mask_v, mask_out)

    return sc_kernel(inds_reac, coeffs_buf)


def _tc_select_body(vals_ref, mask_ref, x_ref, o_ref):
    o_ref[...] = jnp.where(mask_ref[...] != 0.0, vals_ref[...], x_ref[...])


def _tc_select(coeffs, vals_row, mask_row, blk_b):
    B, R = coeffs.shape
    grid = (B // blk_b,)
    return pl.pallas_call(
        _tc_select_body,
        grid=grid,
        in_specs=[
            pl.BlockSpec((1, R), lambda i: (0, 0)),
            pl.BlockSpec((1, R), lambda i: (0, 0)),
            pl.BlockSpec((blk_b, R), lambda i: (i, 0)),
        ],
        out_specs=pl.BlockSpec((blk_b, R), lambda i: (i, 0)),
        out_shape=jax.ShapeDtypeStruct((B, R), jnp.float32),
    )(vals_row, mask_row, coeffs)


def kernel(coeffs, params_med, coeffs_buf, inds_reac):
    B, R = coeffs.shape
    vals_row, mask_row = _sc_build_rows(inds_reac, coeffs_buf, R)
    return _tc_select(coeffs, vals_row.reshape(1, R), mask_row.reshape(1, R),
                      blk_b=64)


# speedup vs baseline: 1.0187x; 1.0187x over previous
import functools

import jax
import jax.numpy as jnp
from jax import lax
from jax.experimental import pallas as pl
from jax.experimental.pallas import tpu as pltpu
from jax.experimental.pallas import tpu_sc as plsc

_LANES = 16


def _sc_build_rows(inds_reac, coeffs_buf, R):
    NSEL = coeffs_buf.shape[0]
    mesh = plsc.VectorSubcoreMesh(core_axis_name="c", subcore_axis_name="s")

    info = plsc.get_sparse_core_info()
    nw = info.num_cores * info.num_subcores
    cols_per_w = R // nw

    @functools.partial(
        pl.kernel,
        mesh=mesh,
        compiler_params=pltpu.CompilerParams(needs_layout_passes=False),
        out_type=[
            jax.ShapeDtypeStruct((R,), jnp.float32),
            jax.ShapeDtypeStruct((R,), jnp.float32),
        ],
        scratch_types=[
            pltpu.VMEM((NSEL,), jnp.int32),
            pltpu.VMEM((NSEL,), jnp.float32),
            pltpu.VMEM((cols_per_w,), jnp.float32),
            pltpu.VMEM((cols_per_w,), jnp.float32),
        ],
    )
    def sc_kernel(inds_hbm, buf_hbm, vals_out, mask_out,
                  inds_v, buf_v, vals_v, mask_v):
        wid = lax.axis_index("s") * info.num_cores + lax.axis_index("c")
        base = wid * cols_per_w

        pltpu.sync_copy(inds_hbm, inds_v)
        pltpu.sync_copy(buf_hbm, buf_v)

        zeros = jnp.zeros((_LANES,), jnp.float32)

        def zero_body(i, carry):
            vals_v[pl.ds(i * _LANES, _LANES)] = zeros
            mask_v[pl.ds(i * _LANES, _LANES)] = zeros
            return carry

        lax.fori_loop(0, cols_per_w // _LANES, zero_body, 0)

        ones = jnp.ones((_LANES,), jnp.float32)

        def scatter_body(j, carry):
            idx = inds_v[pl.ds(j * _LANES, _LANES)] - base
            val = buf_v[pl.ds(j * _LANES, _LANES)]
            keep = jnp.logical_and(idx >= 0, idx < cols_per_w)
            idx_c = jnp.clip(idx, 0, cols_per_w - 1)
            plsc.store_scatter(vals_v, [idx_c], val, mask=keep)
            plsc.store_scatter(mask_v, [idx_c], ones, mask=keep)
            return carry

        lax.fori_loop(0, NSEL // _LANES, scatter_body, 0)

        pltpu.sync_copy(vals_v, vals_out.at[pl.ds(base, cols_per_w)])
        pltpu.sync_copy(mask_v, mask_out.at[pl.ds(base, cols_per_w)])

    return sc_kernel(inds_reac, coeffs_buf)


def _tc_select_body(vals_ref, mask_ref, x_ref, o_ref):
    o_ref[...] = jnp.where(mask_ref[...] != 0.0, vals_ref[...], x_ref[...])


def _tc_select(coeffs, vals_row, mask_row, blk_b):
    B, R = coeffs.shape
    grid = (B // blk_b,)
    return pl.pallas_call(
        _tc_select_body,
        grid=grid,
        in_specs=[
            pl.BlockSpec((1, R), lambda i: (0, 0)),
            pl.BlockSpec((1, R), lambda i: (0, 0)),
            pl.BlockSpec((blk_b, R), lambda i: (i, 0)),
        ],
        out_specs=pl.BlockSpec((blk_b, R), lambda i: (i, 0)),
        out_shape=jax.ShapeDtypeStruct((B, R), jnp.float32),
        compiler_params=pltpu.CompilerParams(
            vmem_limit_bytes=100 * 1024 * 1024),
    )(vals_row, mask_row, coeffs)


def kernel(coeffs, params_med, coeffs_buf, inds_reac):
    B, R = coeffs.shape
    vals_row, mask_row = _sc_build_rows(inds_reac, coeffs_buf, R)
    return _tc_select(coeffs, vals_row.reshape(1, R), mask_row.reshape(1, R),
                      blk_b=128)
